# speedup vs baseline: 5.0358x; 5.0358x over previous
import functools

import jax
import jax.numpy as jnp
from jax import lax
from jax.experimental import pallas as pl
from jax.experimental.pallas import tpu as pltpu
from jax.experimental.pallas import tpu_sc as plsc

D = 128
EPS = 1e-12
NW = 32
CHUNK = 128


@functools.lru_cache(maxsize=None)
def _make_sc_gather(n_chunks: int):
    tok = NW * n_chunks * CHUNK
    per_w = n_chunks * CHUNK
    mesh = plsc.VectorSubcoreMesh(core_axis_name="c", subcore_axis_name="s")

    @functools.partial(
        pl.kernel, mesh=mesh,
        out_type=jax.ShapeDtypeStruct((tok, D), jnp.float32),
        scratch_types=[
            pltpu.VMEM((n_chunks, CHUNK), jnp.int32),
            pltpu.VMEM((CHUNK, D), jnp.float32),
            pltpu.VMEM((CHUNK, D), jnp.float32),
            pltpu.SemaphoreType.DMA,
            pltpu.SemaphoreType.DMA,
        ],
    )
    def sc_gather(ids_hbm, table_hbm, out_hbm, idx_v, buf0, buf1, sem0, sem1):
        wid = lax.axis_index("s") * 2 + lax.axis_index("c")
        base = wid * per_w
        pltpu.sync_copy(ids_hbm.at[wid], idx_v)

        def body(j, carry):
            pltpu.async_copy(table_hbm.at[idx_v.at[j]], buf0, sem0).wait()
            pltpu.sync_copy(buf0, out_hbm.at[pl.ds(base + j * CHUNK, CHUNK)])
            return carry

        lax.fori_loop(0, n_chunks, body, 0)

    return sc_gather


def _tc_body(g, ttf, ctx, pos, te, w, cb, gam, bet, out):
    x = g[...]
    tt = ttf[...][:, :, None]
    t0 = te[0][None, None, :]
    dt = (te[1] - te[0])[None, None, :]
    x = x + pos[...][None, :, :]
    x = x + t0 + tt * dt
    c = ctx[...]
    acc = cb[0][None, None, :]
    for i in range(4):
        acc = acc + c[:, :, i][:, :, None] * w[i][None, None, :]
    x = x + acc
    mu = jnp.mean(x, axis=-1, keepdims=True)
    xc = x - mu
    var = jnp.mean(xc * xc, axis=-1, keepdims=True)
    y = xc * lax.rsqrt(var + EPS)
    out[...] = y * gam[0][None, None, :] + bet[0][None, None, :]


def kernel(input_ids, token_type_ids, context_feature, word_emb, pos_emb,
           type_emb, ctx_W, ctx_b, gamma, beta):
    B, L = input_ids.shape
    tok = B * L
    assert tok % (NW * CHUNK) == 0
    n_chunks = tok // (NW * CHUNK)

    ids3 = input_ids.astype(jnp.int32).reshape(NW, n_chunks, CHUNK)
    g = _make_sc_gather(n_chunks)(ids3, word_emb)
    g = g.reshape(B, L, D)

    ttf = token_type_ids.astype(jnp.float32)
    pos50 = pos_emb[:L]
    cb2 = ctx_b.reshape(1, D)
    gam2 = gamma.reshape(1, D)
    bet2 = beta.reshape(1, D)

    BB = 128
    grid = (B // BB,)
    out = pl.pallas_call(
        _tc_body,
        out_shape=jax.ShapeDtypeStruct((B, L, D), jnp.float32),
        grid=grid,
        in_specs=[
            pl.BlockSpec((BB, L, D), lambda i: (i, 0, 0)),
            pl.BlockSpec((BB, L), lambda i: (i, 0)),
            pl.BlockSpec((BB, L, 4), lambda i: (i, 0, 0)),
            pl.BlockSpec((L, D), lambda i: (0, 0)),
            pl.BlockSpec((2, D), lambda i: (0, 0)),
            pl.BlockSpec((4, D), lambda i: (0, 0)),
            pl.BlockSpec((1, D), lambda i: (0, 0)),
            pl.BlockSpec((1, D), lambda i: (0, 0)),
            pl.BlockSpec((1, D), lambda i: (0, 0)),
        ],
        out_specs=pl.BlockSpec((BB, L, D), lambda i: (i, 0, 0)),
    )(g, ttf, context_feature, pos50, type_emb, ctx_W, cb2, gam2, bet2)
    return out


# speedup vs baseline: 7.8385x; 1.5566x over previous
import functools

import jax
import jax.numpy as jnp
from jax import lax
from jax.experimental import pallas as pl
from jax.experimental.pallas import tpu as pltpu
from jax.experimental.pallas import tpu_sc as plsc

D = 128
L = 50
LP = 56
EPS = 1e-12
NW = 32
BB = 128


@functools.lru_cache(maxsize=None)
def _make_sc_gather(rows_per_w: int):
    b = NW * rows_per_w
    mesh = plsc.VectorSubcoreMesh(core_axis_name="c", subcore_axis_name="s")

    @functools.partial(
        pl.kernel, mesh=mesh,
        out_type=jax.ShapeDtypeStruct((b * LP, D), jnp.float32),
        scratch_types=[
            pltpu.VMEM((rows_per_w, L), jnp.int32),
            pltpu.VMEM((LP, D), jnp.float32),
            pltpu.VMEM((LP, D), jnp.float32),
            pltpu.SemaphoreType.DMA,
            pltpu.SemaphoreType.DMA,
        ],
    )
    def sc_gather(ids_hbm, table_hbm, out_hbm, idx_v, buf0, buf1, sem0, sem1):
        wid = lax.axis_index("s") * 2 + lax.axis_index("c")
        row0 = wid * rows_per_w
        pltpu.sync_copy(ids_hbm.at[wid], idx_v)
        dst0 = buf0.at[pl.ds(0, L)]
        dst1 = buf1.at[pl.ds(0, L)]
        pltpu.async_copy(table_hbm.at[idx_v.at[0]], dst0, sem0)
        n2 = rows_per_w // 2

        def pair(jj, carry):
            j0 = 2 * jj
            pltpu.async_copy(table_hbm.at[idx_v.at[j0 + 1]], dst1, sem1)
            pltpu.make_async_copy(table_hbm.at[idx_v.at[j0]], dst0, sem0).wait()
            pltpu.sync_copy(buf0, out_hbm.at[pl.ds((row0 + j0) * LP, LP)])

            @pl.when(jj < n2 - 1)
            def _():
                pltpu.async_copy(table_hbm.at[idx_v.at[j0 + 2]], dst0, sem0)

            pltpu.make_async_copy(table_hbm.at[idx_v.at[j0 + 1]], dst1, sem1).wait()
            pltpu.sync_copy(buf1, out_hbm.at[pl.ds((row0 + j0 + 1) * LP, LP)])
            return carry

        lax.fori_loop(0, n2, pair, 0)

    return sc_gather


def _tc_body(g, ttf, ctx, pos, w5, bias, gam, bet, out):
    x = g[:, :L, :] + pos[...][None, :, :]
    c5 = jnp.concatenate([ctx[...], ttf[...][:, :, None]], axis=-1)
    proj = lax.dot_general(c5, w5[...], (((2,), (0,)), ((), ())),
                           preferred_element_type=jnp.float32)
    x = x + proj + bias[0][None, None, :]
    mu = jnp.mean(x, axis=-1, keepdims=True)
    xc = x - mu
    var = jnp.mean(xc * xc, axis=-1, keepdims=True)
    y = xc * lax.rsqrt(var + EPS)
    out[...] = y * gam[0][None, None, :] + bet[0][None, None, :]


def kernel(input_ids, token_type_ids, context_feature, word_emb, pos_emb,
           type_emb, ctx_W, ctx_b, gamma, beta):
    B, Lx = input_ids.shape
    assert Lx == L and B % NW == 0
    rows_per_w = B // NW

    ids3 = input_ids.astype(jnp.int32).reshape(NW, rows_per_w, L)
    g2 = _make_sc_gather(rows_per_w)(ids3, word_emb)
    g3 = g2.reshape(B, LP, D)

    ttf = token_type_ids.astype(jnp.float32)
    pos50 = pos_emb[:L]
    w5 = jnp.concatenate([ctx_W, (type_emb[1] - type_emb[0])[None, :]], axis=0)
    bias = (ctx_b + type_emb[0]).reshape(1, D)
    gam2 = gamma.reshape(1, D)
    bet2 = beta.reshape(1, D)

    out = pl.pallas_call(
        _tc_body,
        out_shape=jax.ShapeDtypeStruct((B, L, D), jnp.float32),
        grid=(B // BB,),
        in_specs=[
            pl.BlockSpec((BB, LP, D), lambda i: (i, 0, 0)),
            pl.BlockSpec((BB, L), lambda i: (i, 0)),
            pl.BlockSpec((BB, L, 4), lambda i: (i, 0, 0)),
            pl.BlockSpec((L, D), lambda i: (0, 0)),
            pl.BlockSpec((5, D), lambda i: (0, 0)),
            pl.BlockSpec((1, D), lambda i: (0, 0)),
            pl.BlockSpec((1, D), lambda i: (0, 0)),
            pl.BlockSpec((1, D), lambda i: (0, 0)),
        ],
        out_specs=pl.BlockSpec((BB, L, D), lambda i: (i, 0, 0)),
    )(g3, ttf, context_feature, pos50, w5, bias, gam2, bet2)
    return out


# speedup vs baseline: 14.0392x; 1.7911x over previous
import functools

import jax
import jax.numpy as jnp
from jax import lax
from jax.experimental import pallas as pl
from jax.experimental.pallas import tpu as pltpu
from jax.experimental.pallas import tpu_sc as plsc

D = 128
L = 50
EPS = 1e-12
NW = 32
CHUNK = 128
BBC = 128


@functools.lru_cache(maxsize=None)
def _make_sc_gather(n_chunks: int):
    tok = NW * n_chunks * CHUNK
    per_w = n_chunks * CHUNK
    mesh = plsc.VectorSubcoreMesh(core_axis_name="c", subcore_axis_name="s")

    @functools.partial(
        pl.kernel, mesh=mesh,
        out_type=jax.ShapeDtypeStruct((tok, D), jnp.float32),
        scratch_types=[
            pltpu.VMEM((n_chunks, CHUNK), jnp.int32),
            pltpu.VMEM((CHUNK, D), jnp.float32),
            pltpu.VMEM((CHUNK, D), jnp.float32),
            pltpu.SemaphoreType.DMA,
            pltpu.SemaphoreType.DMA,
        ],
    )
    def sc_gather(ids_hbm, table_hbm, out_hbm, idx_v, buf0, buf1, sem0, sem1):
        wid = lax.axis_index("s") * 2 + lax.axis_index("c")
        base = wid * per_w
        pltpu.sync_copy(ids_hbm.at[wid], idx_v)
        pltpu.async_copy(table_hbm.at[idx_v.at[0]], buf0, sem0)
        n2 = n_chunks // 2

        def pair(jj, carry):
            j0 = 2 * jj
            pltpu.async_copy(table_hbm.at[idx_v.at[j0 + 1]], buf1, sem1)
            pltpu.make_async_copy(table_hbm.at[idx_v.at[j0]], buf0, sem0).wait()
            pltpu.sync_copy(buf0, out_hbm.at[pl.ds(base + j0 * CHUNK, CHUNK)])

            @pl.when(jj < n2 - 1)
            def _():
                pltpu.async_copy(table_hbm.at[idx_v.at[j0 + 2]], buf0, sem0)

            pltpu.make_async_copy(table_hbm.at[idx_v.at[j0 + 1]], buf1, sem1).wait()
            pltpu.sync_copy(buf1, out_hbm.at[pl.ds(base + (j0 + 1) * CHUNK, CHUNK)])
            return carry

        lax.fori_loop(0, n2, pair, 0)

    return sc_gather


def _tc_body(g, ttf, ctx, pos, w5, bias, gam, bet, out):
    x = g[...] + pos[...][:, None, :]
    c5 = jnp.concatenate([ctx[...], ttf[...][:, None, :]], axis=1)
    w5b = jnp.broadcast_to(w5[...][None], (L, D, 5))
    projT = lax.dot_general(w5b, c5, (((2,), (1,)), ((0,), (0,))),
                            preferred_element_type=jnp.float32)
    proj = jnp.swapaxes(projT, 1, 2)
    x = x + proj + bias[0][None, None, :]
    mu = jnp.mean(x, axis=-1, keepdims=True)
    xc = x - mu
    var = jnp.mean(xc * xc, axis=-1, keepdims=True)
    y = xc * lax.rsqrt(var + EPS)
    out[...] = y * gam[0][None, None, :] + bet[0][None, None, :]


def kernel(input_ids, token_type_ids, context_feature, word_emb, pos_emb,
           type_emb, ctx_W, ctx_b, gamma, beta):
    B, Lx = input_ids.shape
    tok = B * Lx
    assert Lx == L and tok % (NW * CHUNK) == 0
    n_chunks = tok // (NW * CHUNK)

    idsT = input_ids.astype(jnp.int32).T.reshape(NW, n_chunks, CHUNK)
    g2 = _make_sc_gather(n_chunks)(idsT, word_emb)
    g3 = g2.reshape(L, B, D)

    ttfT = token_type_ids.T.astype(jnp.float32)
    ctx3 = jnp.transpose(context_feature, (1, 2, 0))
    pos50 = pos_emb[:L]
    w5T = jnp.concatenate([ctx_W, (type_emb[1] - type_emb[0])[None, :]],
                          axis=0).T
    bias = (ctx_b + type_emb[0]).reshape(1, D)
    gam2 = gamma.reshape(1, D)
    bet2 = beta.reshape(1, D)

    outT = pl.pallas_call(
        _tc_body,
        out_shape=jax.ShapeDtypeStruct((L, B, D), jnp.float32),
        grid=(B // BBC,),
        in_specs=[
            pl.BlockSpec((L, BBC, D), lambda i: (0, i, 0)),
            pl.BlockSpec((L, BBC), lambda i: (0, i)),
            pl.BlockSpec((L, 4, BBC), lambda i: (0, 0, i)),
            pl.BlockSpec((L, D), lambda i: (0, 0)),
            pl.BlockSpec((D, 5), lambda i: (0, 0)),
            pl.BlockSpec((1, D), lambda i: (0, 0)),
            pl.BlockSpec((1, D), lambda i: (0, 0)),
            pl.BlockSpec((1, D), lambda i: (0, 0)),
        ],
        out_specs=pl.BlockSpec((L, BBC, D), lambda i: (0, i, 0)),
    )(g3, ttfT, ctx3, pos50, w5T, bias, gam2, bet2)
    return jnp.transpose(outT, (1, 0, 2))
